# baseline (speedup 1.0000x reference)
import math
import functools

import jax
import jax.numpy as jnp
from jax import lax
from jax.experimental import pallas as pl
from jax.experimental.pallas import tpu as pltpu
from jax.experimental.pallas import tpu_sc as plsc

H = 1024
N = 16384
DK = 32
R = 2
TOPK = 16
TOP_M = 8
QDIM = 32
PR = 8
_USE_JNP_TOP8 = False
KS_S = 4
KS_T = 4
TAU = 1.0

BT = 128
NT = N
BLKW = 128
NBLK = N // BLKW
NW = 32


def _score_body(q_ref, k_ref, o_ref, m_ref):
    s = jax.lax.dot_general(
        q_ref[0], k_ref[0], (((1,), (0,)), ((), ())),
        preferred_element_type=jnp.float32)
    o_ref[0] = s
    m_ref[0] = jnp.max(s.reshape(BT, NBLK, BLKW), axis=2)


def _scores_and_blockmax(q2, K2, Bsz):
    grid = (2, Bsz // BT)
    return pl.pallas_call(
        _score_body,
        grid=grid,
        in_specs=[
            pl.BlockSpec((1, BT, 2 * DK), lambda s, i: (s, i, 0)),
            pl.BlockSpec((1, 2 * DK, NT), lambda s, i: (s, 0, 0)),
        ],
        out_specs=[
            pl.BlockSpec((1, BT, NT), lambda s, i: (s, i, 0)),
            pl.BlockSpec((1, BT, NBLK), lambda s, i: (s, i, 0)),
        ],
        out_shape=[
            jax.ShapeDtypeStruct((2, Bsz, N), jnp.float32),
            jax.ShapeDtypeStruct((2, Bsz, NBLK), jnp.float32),
        ],
    )(q2, K2)


def _merge16(rv, ri, cv, ci):
    cvr = lax.rev(cv, (0,))
    cir = lax.rev(ci, (0,))
    take = rv >= cvr
    mv = jnp.where(take, rv, cvr)
    mi = jnp.where(take, ri, cir)
    return plsc.sort_key_val(mv, mi)


def _lane_scalar_i32(vec, j):
    io = lax.iota(jnp.int32, 16)
    return jnp.max(jnp.where(io == j, vec, jnp.int32(0)))


def _make_sc_topk(n_rows):
    rows_per_w = n_rows // NW
    mesh = plsc.VectorSubcoreMesh(core_axis_name="c", subcore_axis_name="s")

    @functools.partial(
        pl.kernel, mesh=mesh,
        compiler_params=pltpu.CompilerParams(needs_layout_passes=False),
        out_type=jax.ShapeDtypeStruct((n_rows, 16), jnp.int32),
        scratch_types=[
            pltpu.VMEM((NBLK,), jnp.float32),
            pltpu.VMEM((NBLK,), jnp.float32),
            pltpu.VMEM((16,), jnp.int32),
            pltpu.VMEM((16,), jnp.int32),
            pltpu.VMEM((16, BLKW), jnp.float32),
            pltpu.VMEM((16 * BLKW,), jnp.float32),
            pltpu.VMEM((16 * BLKW,), jnp.int32),
            pltpu.VMEM((16,), jnp.int32),
            pltpu.SemaphoreType.DMA,
            pltpu.SemaphoreType.DMA,
        ],
    )
    def sc_topk(m_hbm, scores_hbm, out_hbm, m_buf0, m_buf1, ri_buf, bidx,
                blocks, candv, candi, ostage, sem, msem):
        wid = lax.axis_index("s") * 2 + lax.axis_index("c")
        io = lax.iota(jnp.int32, 16)
        neginf = jnp.full((16,), -jnp.inf, jnp.float32)
        base_job = wid * rows_per_w

        def run_job(job, m_buf, next_job, next_buf):
            nx = pltpu.async_copy(m_hbm.at[next_job], next_buf, msem)

            rv = neginf
            ri = jnp.zeros((16,), jnp.int32)
            for c in range(NBLK // 16):
                v = m_buf[pl.ds(c * 16, 16)]
                sv, si = plsc.sort_key_val(v, io + c * 16)
                rv, ri = _merge16(rv, ri, sv, si)
            tau = jnp.min(rv)

            bidx[...] = ri + job * NBLK
            ri_buf[...] = ri
            pltpu.async_copy(scores_hbm.at[bidx], blocks, sem).wait()

            off = jnp.int32(0)
            for j in range(16):
                for c in range(BLKW // 16):
                    v = blocks[j, pl.ds(c * 16, 16)]
                    msk = v >= tau
                    cnt = jnp.sum(msk.astype(jnp.int32))
                    plsc.store_compressed(candv.at[pl.ds(off, 16)], v, mask=msk)
                    plsc.store_compressed(
                        candi.at[pl.ds(off, 16)],
                        io + (j * BLKW + c * 16), mask=msk)
                    off = off + cnt

            def cond(carry):
                i, _, _ = carry
                return i < off

            def body(carry):
                i, rv2, ri2 = carry
                v = candv[pl.ds(i, 16)]
                ix = candi[pl.ds(i, 16)]
                valid = (i + io) < off
                v = jnp.where(valid, v, -jnp.inf)
                sv, si = plsc.sort_key_val(v, ix)
                rv2, ri2 = _merge16(rv2, ri2, sv, si)
                return (i + 16, rv2, ri2)

            _, rv2, ri2 = lax.while_loop(
                cond, body, (jnp.int32(0), neginf, jnp.zeros((16,), jnp.int32)))

            loc = lax.rev(ri2, (0,))
            blk = lax.shift_right_logical(loc, 7)
            ostage[...] = (plsc.load_gather(ri_buf, [blk]) * BLKW
                           + jnp.bitwise_and(loc, jnp.int32(BLKW - 1)))
            pltpu.sync_copy(ostage, out_hbm.at[job])
            nx.wait()

        def job_pair(t, _):
            job = base_job + 2 * t
            run_job(job, m_buf0, job + 1, m_buf1)
            nxt = jnp.where(t + 1 < rows_per_w // 2, job + 2, job)
            run_job(job + 1, m_buf1, nxt, m_buf0)
            return _

        pltpu.async_copy(m_hbm.at[base_job], m_buf0, msem).wait()
        lax.fori_loop(0, rows_per_w // 2, job_pair, 0)

    return sc_topk


BT2 = 512


def _top8_body(s_ref, i_ref, w_ref):
    s = s_ref[...]
    io = jax.lax.broadcasted_iota(jnp.int32, s.shape, 1)
    rem = s
    vals, idxs = [], []
    for _ in range(TOP_M):
        vmax = jnp.max(rem, axis=1, keepdims=True)
        cand = jnp.where(rem == vmax, io, jnp.int32(1 << 30))
        amin = jnp.min(cand, axis=1, keepdims=True)
        vals.append(vmax)
        idxs.append(amin)
        rem = jnp.where(io == amin, -jnp.inf, rem)
    v = jnp.concatenate(vals, axis=1)
    ii = jnp.concatenate(idxs, axis=1)
    i_ref[...] = jnp.concatenate(
        [ii, jnp.zeros_like(ii)], axis=1)
    e = jnp.exp(v / TAU - v[:, 0:1] / TAU)
    w_ref[...] = e / jnp.sum(e, axis=1, keepdims=True)


def _top8_weights(S_flat, Bsz):
    return pl.pallas_call(
        _top8_body,
        grid=(Bsz // BT2,),
        in_specs=[pl.BlockSpec((BT2, TOPK * TOPK), lambda i: (i, 0))],
        out_specs=[
            pl.BlockSpec((BT2, 16), lambda i: (i, 0)),
            pl.BlockSpec((BT2, TOP_M), lambda i: (i, 0)),
        ],
        out_shape=[
            jax.ShapeDtypeStruct((Bsz, 16), jnp.int32),
            jax.ShapeDtypeStruct((Bsz, TOP_M), jnp.float32),
        ],
    )(S_flat)


def _make_sc_pick(Bsz):
    rows_per_w = Bsz // NW
    mesh = plsc.VectorSubcoreMesh(core_axis_name="c", subcore_axis_name="s")

    @functools.partial(
        pl.kernel, mesh=mesh,
        compiler_params=pltpu.CompilerParams(
            needs_layout_passes=False, use_tc_tiling_on_sc=False),
        out_type=[
            jax.ShapeDtypeStruct((Bsz, 16, QDIM), jnp.float32),
            jax.ShapeDtypeStruct((Bsz, 16, QDIM), jnp.float32),
        ],
        scratch_types=[
            pltpu.VMEM((16,), jnp.int32),
            pltpu.VMEM((16,), jnp.int32),
            pltpu.VMEM((16,), jnp.int32),
            pltpu.VMEM((16,), jnp.int32),
            pltpu.VMEM((16,), jnp.int32),
            pltpu.VMEM((16, QDIM), jnp.float32),
            pltpu.VMEM((16, QDIM), jnp.float32),
            pltpu.SemaphoreType.DMA,
        ],
    )
    def sc_pick(tidx_hbm, ridx_hbm, cidx_hbm, remb_hbm, cemb_hbm,
                rv_hbm, cv_hbm, tvec, ridx, cidx, pickr, pickc,
                rvec, cvec, sem):
        wid = lax.axis_index("s") * 2 + lax.axis_index("c")

        def job_body(t, _):
            b = wid * rows_per_w + t
            pltpu.sync_copy(tidx_hbm.at[b], tvec)
            pltpu.sync_copy(ridx_hbm.at[b], ridx)
            pltpu.sync_copy(cidx_hbm.at[b], cidx)
            tv = tvec[...]
            rp = lax.shift_right_logical(tv, 4)
            cp = jnp.bitwise_and(tv, jnp.int32(15))
            pickr[...] = plsc.load_gather(ridx, [rp])
            pickc[...] = plsc.load_gather(cidx, [cp])
            c1 = pltpu.async_copy(remb_hbm.at[pickr], rvec, sem)
            c2 = pltpu.async_copy(cemb_hbm.at[pickc], cvec, sem)
            c1.wait()
            c2.wait()
            pltpu.sync_copy(rvec, rv_hbm.at[b])
            pltpu.sync_copy(cvec, cv_hbm.at[b])
            return _

        lax.fori_loop(0, rows_per_w, job_body, 0)

    return sc_pick


BT3 = 512


def _first_occurrence_kth(a, k):
    io = jax.lax.broadcasted_iota(jnp.int32, a.shape, 1)
    rem = a
    vmax = None
    for _ in range(k):
        vmax = jnp.max(rem, axis=1, keepdims=True)
        cand = jnp.where(rem == vmax, io, jnp.int32(1 << 30))
        amin = jnp.min(cand, axis=1, keepdims=True)
        rem = jnp.where(io == amin, -jnp.inf, rem)
    return vmax


def _sparsify_tc(S, k):
    a = jnp.abs(S)
    thresh = _first_occurrence_kth(a, k)
    return jnp.where(a >= thresh, S, jnp.zeros_like(S))


def _tail_body(rv_ref, cv_ref, w_ref, x_ref, rS_ref, cS_ref, rT_ref,
               cT_ref, xu_ref, bm_ref, vp_ref, up_ref, tg_ref, o_ref):
    f32 = jnp.float32
    rv2 = rv_ref[...].reshape(BT3 * TOP_M, QDIM)
    cv2 = cv_ref[...].reshape(BT3 * TOP_M, QDIM)

    def dot(a, b):
        return jax.lax.dot_general(a, b, (((1,), (0,)), ((), ())),
                                   preferred_element_type=f32)

    S = dot(rv2, rS_ref[...]) + dot(cv2, cS_ref[...])
    T = dot(rv2, rT_ref[...]) + dot(cv2, cT_ref[...])
    S = _sparsify_tc(S, KS_S)
    T = _sparsify_tc(T, KS_T)
    u = dot(x_ref[...], xu_ref[...])
    T3 = T.reshape(BT3, TOP_M, QDIM)
    T3b = T3.astype(jnp.bfloat16).astype(f32)
    ub = u.astype(jnp.bfloat16).astype(f32)
    pv = jnp.sum(T3b * ub[:, None, :], axis=2)
    aw = w_ref[...] * pv
    S3 = S.reshape(BT3, TOP_M, QDIM)
    s_acc = jnp.sum(aw[:, :, None] * S3, axis=1)
    nrm = jnp.sqrt(jnp.sum(s_acc * s_acc, axis=1, keepdims=True))
    s_acc = s_acc / jnp.maximum(nrm, 1e-12)
    G = dot(s_acc, bm_ref[...])
    lr = dot(dot(G, vp_ref[...]), up_ref[...])
    o_ref[...] = G + tg_ref[0] * lr


def _tail(row_vecs, col_vecs, weights, x, rS, cS, rT, cT, xu, Bm, vp, up,
          tg, Bsz):
    return pl.pallas_call(
        _tail_body,
        grid=(Bsz // BT3,),
        in_specs=[
            pl.BlockSpec((BT3, TOP_M, QDIM), lambda i: (i, 0, 0)),
            pl.BlockSpec((BT3, TOP_M, QDIM), lambda i: (i, 0, 0)),
            pl.BlockSpec((BT3, TOP_M), lambda i: (i, 0)),
            pl.BlockSpec((BT3, H), lambda i: (i, 0)),
            pl.BlockSpec((QDIM, QDIM), lambda i: (0, 0)),
            pl.BlockSpec((QDIM, QDIM), lambda i: (0, 0)),
            pl.BlockSpec((QDIM, QDIM), lambda i: (0, 0)),
            pl.BlockSpec((QDIM, QDIM), lambda i: (0, 0)),
            pl.BlockSpec((H, QDIM), lambda i: (0, 0)),
            pl.BlockSpec((QDIM, H), lambda i: (0, 0)),
            pl.BlockSpec((H, PR), lambda i: (0, 0)),
            pl.BlockSpec((PR, H), lambda i: (0, 0)),
            pl.BlockSpec(memory_space=pltpu.SMEM),
        ],
        out_specs=pl.BlockSpec((BT3, H), lambda i: (i, 0)),
        out_shape=jax.ShapeDtypeStruct((Bsz, H), jnp.float32),
    )(row_vecs, col_vecs, weights, x, rS, cS, rT, cT, xu, Bm, vp, up, tg)


def kernel(x, q_W, K_row, K_col, core, row_mix, col_mix, row_emb, col_emb,
           row_to_S, col_to_S, row_to_T, col_to_T, Bm, x_to_U, Vproj, Uproj,
           gamma):
    Bsz = x.shape[0]
    q_all = (x @ q_W.T).reshape(Bsz, 2, R, DK)
    qrow = q_all[:, 0]
    qcol = q_all[:, 1]

    KrfT = jnp.transpose(K_row, (1, 0, 2)).reshape(N, R * DK).T
    KcfT = jnp.transpose(K_col, (1, 0, 2)).reshape(N, R * DK).T
    sr = jnp.repeat(row_mix, DK)[:, None]
    sc = jnp.repeat(col_mix, DK)[:, None]

    q2 = jnp.stack([qrow.reshape(Bsz, -1), qcol.reshape(Bsz, -1)])
    K2 = jnp.stack([KrfT * sr, KcfT * sc])
    scores, bmax = _scores_and_blockmax(q2, K2, Bsz)

    idx = _make_sc_topk(2 * Bsz)(
        bmax.reshape(2 * Bsz, NBLK),
        scores.reshape(2 * Bsz * NBLK, BLKW))
    row_idx = idx[:Bsz]
    col_idx = idx[Bsz:]

    K_row_sel = jnp.transpose(jnp.take(K_row, row_idx, axis=1), (1, 0, 2, 3))
    K_col_sel = jnp.transpose(jnp.take(K_col, col_idx, axis=1), (1, 0, 2, 3))
    qrow_mixed = jnp.einsum('ij,brk->bjk', core.T, qrow)
    A_sel = jnp.einsum('brpk,bjk->bjp', K_row_sel, qrow_mixed)
    B_sel = jnp.einsum('brqk,brk->brq', K_col_sel, qcol)
    Sgrid = jnp.einsum('brp,brn->bpn', A_sel, B_sel)

    S_flat = Sgrid.reshape(Bsz, TOPK * TOPK)
    if _USE_JNP_TOP8:
        top_scores, top_idx = jax.lax.top_k(S_flat, TOP_M)
        weights = jax.nn.softmax(top_scores / TAU, axis=1)
        top_idx16 = jnp.concatenate(
            [top_idx, jnp.zeros_like(top_idx)], axis=1)
    else:
        top_idx16, weights = _top8_weights(S_flat, Bsz)

    rv16, cv16 = _make_sc_pick(Bsz)(
        top_idx16, row_idx, col_idx, row_emb, col_emb)
    row_vecs = rv16[:, :TOP_M]
    col_vecs = cv16[:, :TOP_M]

    tg = jnp.tanh(gamma).reshape(1)
    return _tail(row_vecs, col_vecs, weights, x,
                 row_to_S.T, col_to_S.T, row_to_T.T, col_to_T.T,
                 x_to_U.T, Bm, Vproj.T, Uproj.T, tg, Bsz)


# speedup vs baseline: 1.0289x; 1.0289x over previous
import math
import functools

import jax
import jax.numpy as jnp
from jax import lax
from jax.experimental import pallas as pl
from jax.experimental.pallas import tpu as pltpu
from jax.experimental.pallas import tpu_sc as plsc

H = 1024
N = 16384
DK = 32
R = 2
TOPK = 16
TOP_M = 8
QDIM = 32
PR = 8
_USE_JNP_TOP8 = False
KS_S = 4
KS_T = 4
TAU = 1.0

BT = 128
NT = N
BLKW = 128
NBLK = N // BLKW
NW = 32


def _score_body(q_ref, k_ref, o_ref, m_ref):
    s = jax.lax.dot_general(
        q_ref[0], k_ref[0], (((1,), (0,)), ((), ())),
        preferred_element_type=jnp.float32)
    o_ref[0] = s
    m_ref[0] = jnp.max(s.reshape(BT, NBLK, BLKW), axis=2)


def _scores_and_blockmax(q2, K2, Bsz):
    grid = (2, Bsz // BT)
    return pl.pallas_call(
        _score_body,
        grid=grid,
        in_specs=[
            pl.BlockSpec((1, BT, 2 * DK), lambda s, i: (s, i, 0)),
            pl.BlockSpec((1, 2 * DK, NT), lambda s, i: (s, 0, 0)),
        ],
        out_specs=[
            pl.BlockSpec((1, BT, NT), lambda s, i: (s, i, 0)),
            pl.BlockSpec((1, BT, NBLK), lambda s, i: (s, i, 0)),
        ],
        out_shape=[
            jax.ShapeDtypeStruct((2, Bsz, N), jnp.float32),
            jax.ShapeDtypeStruct((2, Bsz, NBLK), jnp.float32),
        ],
    )(q2, K2)


def _merge16(rv, ri, cv, ci):
    cvr = lax.rev(cv, (0,))
    cir = lax.rev(ci, (0,))
    take = rv >= cvr
    mv = jnp.where(take, rv, cvr)
    mi = jnp.where(take, ri, cir)
    return plsc.sort_key_val(mv, mi)


def _lane_scalar_i32(vec, j):
    io = lax.iota(jnp.int32, 16)
    return jnp.max(jnp.where(io == j, vec, jnp.int32(0)))


def _make_sc_topk(n_rows):
    rows_per_w = n_rows // NW
    mesh = plsc.VectorSubcoreMesh(core_axis_name="c", subcore_axis_name="s")

    @functools.partial(
        pl.kernel, mesh=mesh,
        compiler_params=pltpu.CompilerParams(needs_layout_passes=False),
        out_type=jax.ShapeDtypeStruct((n_rows, 16), jnp.int32),
        scratch_types=[
            pltpu.VMEM((NBLK,), jnp.float32),
            pltpu.VMEM((NBLK,), jnp.float32),
            pltpu.VMEM((16,), jnp.int32),
            pltpu.VMEM((16,), jnp.int32),
            pltpu.VMEM((16, BLKW), jnp.float32),
            pltpu.VMEM((16 * BLKW,), jnp.float32),
            pltpu.VMEM((16 * BLKW,), jnp.int32),
            pltpu.VMEM((16,), jnp.int32),
            pltpu.SemaphoreType.DMA,
            pltpu.SemaphoreType.DMA,
        ],
    )
    def sc_topk(m_hbm, scores_hbm, out_hbm, m_buf0, m_buf1, ri_buf, bidx,
                blocks, candv, candi, ostage, sem, msem):
        wid = lax.axis_index("s") * 2 + lax.axis_index("c")
        io = lax.iota(jnp.int32, 16)
        neginf = jnp.full((16,), -jnp.inf, jnp.float32)
        base_job = wid * rows_per_w

        def run_job(job, m_buf, next_job, next_buf):
            nx = pltpu.async_copy(m_hbm.at[next_job], next_buf, msem)

            rv = neginf
            ri = jnp.zeros((16,), jnp.int32)
            for c in range(NBLK // 16):
                v = m_buf[pl.ds(c * 16, 16)]
                sv, si = plsc.sort_key_val(v, io + c * 16)
                rv, ri = _merge16(rv, ri, sv, si)
            tau = jnp.min(rv)

            bidx[...] = ri + job * NBLK
            ri_buf[...] = ri
            pltpu.async_copy(scores_hbm.at[bidx], blocks, sem).wait()

            off = jnp.int32(0)
            for j in range(16):
                for c in range(BLKW // 16):
                    v = blocks[j, pl.ds(c * 16, 16)]
                    msk = v >= tau
                    cnt = plsc.all_reduce_population_count(msk)[0]
                    plsc.store_compressed(candv.at[pl.ds(off, 16)], v, mask=msk)
                    plsc.store_compressed(
                        candi.at[pl.ds(off, 16)],
                        io + (j * BLKW + c * 16), mask=msk)
                    off = off + cnt

            def cond(carry):
                i, _, _ = carry
                return i < off

            def body(carry):
                i, rv2, ri2 = carry
                v = candv[pl.ds(i, 16)]
                ix = candi[pl.ds(i, 16)]
                valid = (i + io) < off
                v = jnp.where(valid, v, -jnp.inf)
                sv, si = plsc.sort_key_val(v, ix)
                rv2, ri2 = _merge16(rv2, ri2, sv, si)
                return (i + 16, rv2, ri2)

            _, rv2, ri2 = lax.while_loop(
                cond, body, (jnp.int32(0), neginf, jnp.zeros((16,), jnp.int32)))

            loc = lax.rev(ri2, (0,))
            blk = lax.shift_right_logical(loc, 7)
            ostage[...] = (plsc.load_gather(ri_buf, [blk]) * BLKW
                           + jnp.bitwise_and(loc, jnp.int32(BLKW - 1)))
            pltpu.sync_copy(ostage, out_hbm.at[job])
            nx.wait()

        def job_pair(t, _):
            job = base_job + 2 * t
            run_job(job, m_buf0, job + 1, m_buf1)
            nxt = jnp.where(t + 1 < rows_per_w // 2, job + 2, job)
            run_job(job + 1, m_buf1, nxt, m_buf0)
            return _

        pltpu.async_copy(m_hbm.at[base_job], m_buf0, msem).wait()
        lax.fori_loop(0, rows_per_w // 2, job_pair, 0)

    return sc_topk


BT2 = 512


def _top8_body(s_ref, i_ref, w_ref):
    s = s_ref[...]
    io = jax.lax.broadcasted_iota(jnp.int32, s.shape, 1)
    rem = s
    vals, idxs = [], []
    for _ in range(TOP_M):
        vmax = jnp.max(rem, axis=1, keepdims=True)
        cand = jnp.where(rem == vmax, io, jnp.int32(1 << 30))
        amin = jnp.min(cand, axis=1, keepdims=True)
        vals.append(vmax)
        idxs.append(amin)
        rem = jnp.where(io == amin, -jnp.inf, rem)
    v = jnp.concatenate(vals, axis=1)
    ii = jnp.concatenate(idxs, axis=1)
    i_ref[...] = jnp.concatenate(
        [ii, jnp.zeros_like(ii)], axis=1)
    e = jnp.exp(v / TAU - v[:, 0:1] / TAU)
    w_ref[...] = e / jnp.sum(e, axis=1, keepdims=True)


def _top8_weights(S_flat, Bsz):
    return pl.pallas_call(
        _top8_body,
        grid=(Bsz // BT2,),
        in_specs=[pl.BlockSpec((BT2, TOPK * TOPK), lambda i: (i, 0))],
        out_specs=[
            pl.BlockSpec((BT2, 16), lambda i: (i, 0)),
            pl.BlockSpec((BT2, TOP_M), lambda i: (i, 0)),
        ],
        out_shape=[
            jax.ShapeDtypeStruct((Bsz, 16), jnp.int32),
            jax.ShapeDtypeStruct((Bsz, TOP_M), jnp.float32),
        ],
    )(S_flat)


def _make_sc_pick(Bsz):
    rows_per_w = Bsz // NW
    mesh = plsc.VectorSubcoreMesh(core_axis_name="c", subcore_axis_name="s")

    @functools.partial(
        pl.kernel, mesh=mesh,
        compiler_params=pltpu.CompilerParams(
            needs_layout_passes=False, use_tc_tiling_on_sc=False),
        out_type=[
            jax.ShapeDtypeStruct((Bsz, 16, QDIM), jnp.float32),
            jax.ShapeDtypeStruct((Bsz, 16, QDIM), jnp.float32),
        ],
        scratch_types=[
            pltpu.VMEM((16,), jnp.int32),
            pltpu.VMEM((16,), jnp.int32),
            pltpu.VMEM((16,), jnp.int32),
            pltpu.VMEM((16,), jnp.int32),
            pltpu.VMEM((16,), jnp.int32),
            pltpu.VMEM((16, QDIM), jnp.float32),
            pltpu.VMEM((16, QDIM), jnp.float32),
            pltpu.SemaphoreType.DMA,
        ],
    )
    def sc_pick(tidx_hbm, ridx_hbm, cidx_hbm, remb_hbm, cemb_hbm,
                rv_hbm, cv_hbm, tvec, ridx, cidx, pickr, pickc,
                rvec, cvec, sem):
        wid = lax.axis_index("s") * 2 + lax.axis_index("c")

        def job_body(t, _):
            b = wid * rows_per_w + t
            pltpu.sync_copy(tidx_hbm.at[b], tvec)
            pltpu.sync_copy(ridx_hbm.at[b], ridx)
            pltpu.sync_copy(cidx_hbm.at[b], cidx)
            tv = tvec[...]
            rp = lax.shift_right_logical(tv, 4)
            cp = jnp.bitwise_and(tv, jnp.int32(15))
            pickr[...] = plsc.load_gather(ridx, [rp])
            pickc[...] = plsc.load_gather(cidx, [cp])
            c1 = pltpu.async_copy(remb_hbm.at[pickr], rvec, sem)
            c2 = pltpu.async_copy(cemb_hbm.at[pickc], cvec, sem)
            c1.wait()
            c2.wait()
            pltpu.sync_copy(rvec, rv_hbm.at[b])
            pltpu.sync_copy(cvec, cv_hbm.at[b])
            return _

        lax.fori_loop(0, rows_per_w, job_body, 0)

    return sc_pick


BT3 = 512


def _first_occurrence_kth(a, k):
    io = jax.lax.broadcasted_iota(jnp.int32, a.shape, 1)
    rem = a
    vmax = None
    for _ in range(k):
        vmax = jnp.max(rem, axis=1, keepdims=True)
        cand = jnp.where(rem == vmax, io, jnp.int32(1 << 30))
        amin = jnp.min(cand, axis=1, keepdims=True)
        rem = jnp.where(io == amin, -jnp.inf, rem)
    return vmax


def _sparsify_tc(S, k):
    a = jnp.abs(S)
    thresh = _first_occurrence_kth(a, k)
    return jnp.where(a >= thresh, S, jnp.zeros_like(S))


def _tail_body(rv_ref, cv_ref, w_ref, x_ref, rS_ref, cS_ref, rT_ref,
               cT_ref, xu_ref, bm_ref, vp_ref, up_ref, tg_ref, o_ref):
    f32 = jnp.float32
    rv2 = rv_ref[...].reshape(BT3 * TOP_M, QDIM)
    cv2 = cv_ref[...].reshape(BT3 * TOP_M, QDIM)

    def dot(a, b):
        return jax.lax.dot_general(a, b, (((1,), (0,)), ((), ())),
                                   preferred_element_type=f32)

    S = dot(rv2, rS_ref[...]) + dot(cv2, cS_ref[...])
    T = dot(rv2, rT_ref[...]) + dot(cv2, cT_ref[...])
    S = _sparsify_tc(S, KS_S)
    T = _sparsify_tc(T, KS_T)
    u = dot(x_ref[...], xu_ref[...])
    T3 = T.reshape(BT3, TOP_M, QDIM)
    T3b = T3.astype(jnp.bfloat16).astype(f32)
    ub = u.astype(jnp.bfloat16).astype(f32)
    pv = jnp.sum(T3b * ub[:, None, :], axis=2)
    aw = w_ref[...] * pv
    S3 = S.reshape(BT3, TOP_M, QDIM)
    s_acc = jnp.sum(aw[:, :, None] * S3, axis=1)
    nrm = jnp.sqrt(jnp.sum(s_acc * s_acc, axis=1, keepdims=True))
    s_acc = s_acc / jnp.maximum(nrm, 1e-12)
    G = dot(s_acc, bm_ref[...])
    lr = dot(dot(G, vp_ref[...]), up_ref[...])
    o_ref[...] = G + tg_ref[0] * lr


def _tail(row_vecs, col_vecs, weights, x, rS, cS, rT, cT, xu, Bm, vp, up,
          tg, Bsz):
    return pl.pallas_call(
        _tail_body,
        grid=(Bsz // BT3,),
        in_specs=[
            pl.BlockSpec((BT3, TOP_M, QDIM), lambda i: (i, 0, 0)),
            pl.BlockSpec((BT3, TOP_M, QDIM), lambda i: (i, 0, 0)),
            pl.BlockSpec((BT3, TOP_M), lambda i: (i, 0)),
            pl.BlockSpec((BT3, H), lambda i: (i, 0)),
            pl.BlockSpec((QDIM, QDIM), lambda i: (0, 0)),
            pl.BlockSpec((QDIM, QDIM), lambda i: (0, 0)),
            pl.BlockSpec((QDIM, QDIM), lambda i: (0, 0)),
            pl.BlockSpec((QDIM, QDIM), lambda i: (0, 0)),
            pl.BlockSpec((H, QDIM), lambda i: (0, 0)),
            pl.BlockSpec((QDIM, H), lambda i: (0, 0)),
            pl.BlockSpec((H, PR), lambda i: (0, 0)),
            pl.BlockSpec((PR, H), lambda i: (0, 0)),
            pl.BlockSpec(memory_space=pltpu.SMEM),
        ],
        out_specs=pl.BlockSpec((BT3, H), lambda i: (i, 0)),
        out_shape=jax.ShapeDtypeStruct((Bsz, H), jnp.float32),
    )(row_vecs, col_vecs, weights, x, rS, cS, rT, cT, xu, Bm, vp, up, tg)


def kernel(x, q_W, K_row, K_col, core, row_mix, col_mix, row_emb, col_emb,
           row_to_S, col_to_S, row_to_T, col_to_T, Bm, x_to_U, Vproj, Uproj,
           gamma):
    Bsz = x.shape[0]
    q_all = (x @ q_W.T).reshape(Bsz, 2, R, DK)
    qrow = q_all[:, 0]
    qcol = q_all[:, 1]

    KrfT = jnp.transpose(K_row, (1, 0, 2)).reshape(N, R * DK).T
    KcfT = jnp.transpose(K_col, (1, 0, 2)).reshape(N, R * DK).T
    sr = jnp.repeat(row_mix, DK)[:, None]
    sc = jnp.repeat(col_mix, DK)[:, None]

    q2 = jnp.stack([qrow.reshape(Bsz, -1), qcol.reshape(Bsz, -1)])
    K2 = jnp.stack([KrfT * sr, KcfT * sc])
    scores, bmax = _scores_and_blockmax(q2, K2, Bsz)

    idx = _make_sc_topk(2 * Bsz)(
        bmax.reshape(2 * Bsz, NBLK),
        scores.reshape(2 * Bsz * NBLK, BLKW))
    row_idx = idx[:Bsz]
    col_idx = idx[Bsz:]

    K_row_sel = jnp.transpose(jnp.take(K_row, row_idx, axis=1), (1, 0, 2, 3))
    K_col_sel = jnp.transpose(jnp.take(K_col, col_idx, axis=1), (1, 0, 2, 3))
    qrow_mixed = jnp.einsum('ij,brk->bjk', core.T, qrow)
    A_sel = jnp.einsum('brpk,bjk->bjp', K_row_sel, qrow_mixed)
    B_sel = jnp.einsum('brqk,brk->brq', K_col_sel, qcol)
    Sgrid = jnp.einsum('brp,brn->bpn', A_sel, B_sel)

    S_flat = Sgrid.reshape(Bsz, TOPK * TOPK)
    if _USE_JNP_TOP8:
        top_scores, top_idx = jax.lax.top_k(S_flat, TOP_M)
        weights = jax.nn.softmax(top_scores / TAU, axis=1)
        top_idx16 = jnp.concatenate(
            [top_idx, jnp.zeros_like(top_idx)], axis=1)
    else:
        top_idx16, weights = _top8_weights(S_flat, Bsz)

    rv16, cv16 = _make_sc_pick(Bsz)(
        top_idx16, row_idx, col_idx, row_emb, col_emb)
    row_vecs = rv16[:, :TOP_M]
    col_vecs = cv16[:, :TOP_M]

    tg = jnp.tanh(gamma).reshape(1)
    return _tail(row_vecs, col_vecs, weights, x,
                 row_to_S.T, col_to_S.T, row_to_T.T, col_to_T.T,
                 x_to_U.T, Bm, Vproj.T, Uproj.T, tg, Bsz)
